# Initial kernel scaffold; baseline (speedup 1.0000x reference)
import jax
import jax.numpy as jnp
from jax.experimental import pallas as pl


def kernel(ques_feat, his_feat, rcnn_feat, ques_emb, params, rnd):
    raise NotImplementedError("write your pallas kernel here")


# speedup vs baseline: 15.0260x; 15.0260x over previous
import functools

import jax
import jax.numpy as jnp
from jax.experimental import pallas as pl
from jax.experimental.pallas import tpu as pltpu

_NHID = 512
_L = 20
_RND = 10
_K = 36
_NB = 8

_MAT_W = ["Wq_1", "Wh_1", "ref_att", "ref_att2", "ref_att3",
          "W3", "W4", "W5", "W6", "W7", "W8", "W9", "W10", "fc1"]
_ROW_W = ["Wa_1", "Wqt", "Wqt2", "Wqt3", "W11"]


def _bmm(a, b):
    return jax.lax.dot_general(
        a, b, (((2,), (1,)), ((0,), (0,))), preferred_element_type=jnp.float32)


def _top8_weights(belief):
    neg = jnp.float32(-1e30)
    iota = jax.lax.broadcasted_iota(jnp.int32, belief.shape, 2)
    vals = belief
    mask = jnp.zeros(belief.shape, jnp.bool_)
    for _ in range(_NB):
        m = jnp.max(vals, axis=-1, keepdims=True)
        cand = vals >= m
        idx = jnp.min(jnp.where(cand, iota, _K), axis=-1, keepdims=True)
        one = iota == idx
        mask = jnp.logical_or(mask, one)
        vals = jnp.where(one, neg, vals)
    mx = jnp.max(belief, axis=-1, keepdims=True)
    e = jnp.where(mask, jnp.exp(belief - mx), 0.0)
    return e / jnp.sum(e, axis=-1, keepdims=True)


def _fused(refs):
    (qlast_ref, his_ref, rcnn_ref, qt_ref, qet_ref, w_refs, b_refs,
     rw_refs, rb_refs, out_ref) = refs
    f32 = jnp.float32
    qlast = qlast_ref[...]
    his = his_ref[...]
    rcnn = rcnn_ref[...]
    qt = qt_ref[...]
    qet = qet_ref[...]
    bB = qlast.shape[0]

    W = {n: w_refs[i][...] for i, n in enumerate(_MAT_W)}
    Bv = {n: b_refs[i][...] for i, n in enumerate(_MAT_W)}
    RW = {n: rw_refs[i][...] for i, n in enumerate(_ROW_W)}
    RB = {n: rb_refs[i][...] for i, n in enumerate(_ROW_W)}

    def lin(x, n):
        return jnp.dot(x, W[n], preferred_element_type=f32) + Bv[n]

    def rowlin3(x, n):
        return jnp.sum(x * RW[n][None, :, :], axis=-1) + RB[n][0, 0]

    def softmax(x):
        m = jnp.max(x, axis=-1, keepdims=True)
        e = jnp.exp(x - m)
        return e / jnp.sum(e, axis=-1, keepdims=True)

    q1 = lin(qlast, "Wq_1")
    he = lin(his.reshape(bB * _RND, _NHID), "Wh_1").reshape(bB, _RND, _NHID)
    att1 = jnp.tanh(he + q1[:, None, :])
    haw = softmax(rowlin3(att1, "Wa_1"))
    h_emb = jnp.sum(haw[:, :, None] * his, axis=1)
    h_exp = jnp.broadcast_to(h_emb[:, None, :], (bB, _K, _NHID))

    def q_ctx(ref_name, wqt_name):
        qn = jax.nn.sigmoid(
            lin(qt.reshape(bB * _L, _NHID), ref_name)).reshape(bB, _L, _NHID)
        nrm = jnp.sqrt(jnp.sum(qn * qn, axis=-1, keepdims=True))
        qn = qn / jnp.maximum(nrm, 1e-12)
        at = softmax(rowlin3(qn, wqt_name))
        return jnp.sum(at[:, :, None] * qet, axis=1)

    def round_fn(g, qc):
        gf = g.reshape(bB * _K, 2 * _NHID)
        mes_b = (lin(gf, "W3").reshape(bB, _K, _NHID)
                 * lin(qc, "W4")[:, None, :])
        g5 = lin(gf, "W5").reshape(bB, _K, _NHID)
        belief = _bmm(g5, jnp.transpose(mes_b, (0, 2, 1)))
        mes = (lin(gf, "W6").reshape(bB, _K, _NHID)
               * lin(qc, "W7")[:, None, :])
        w8 = _top8_weights(belief)
        return _bmm(w8, mes)

    qc1 = q_ctx("ref_att", "Wqt")
    sm1 = round_fn(jnp.concatenate((rcnn, h_exp), axis=2), qc1)
    ctx1 = lin(jnp.concatenate((h_exp, sm1), axis=2).reshape(bB * _K, 2 * _NHID),
               "W8").reshape(bB, _K, _NHID)

    qc2 = q_ctx("ref_att2", "Wqt2")
    sm2 = round_fn(jnp.concatenate((rcnn, ctx1), axis=2), qc2)
    ctx2 = lin(jnp.concatenate((ctx1, sm2), axis=2).reshape(bB * _K, 2 * _NHID),
               "W8").reshape(bB, _K, _NHID)

    qc3 = q_ctx("ref_att3", "Wqt3")
    sm3 = round_fn(jnp.concatenate((rcnn, ctx2), axis=2), qc3)
    ctx3 = lin(jnp.concatenate((ctx2, sm3), axis=2).reshape(bB * _K, 2 * _NHID),
               "W8").reshape(bB, _K, _NHID)

    graph4 = jnp.concatenate((rcnn, ctx3), axis=2)
    g2 = lin(graph4.reshape(bB * _K, 2 * _NHID), "W9").reshape(bB, _K, _NHID)
    qe2 = lin(qlast, "W10")
    attg = jnp.tanh(g2 + qe2[:, None, :])
    gatt = softmax(rowlin3(attg, "W11"))
    graph_emb = jnp.sum(gatt[:, :, None] * graph4, axis=1)
    concat_feat = jnp.concatenate((graph_emb, qlast, h_emb), axis=1)
    out_ref[...] = jnp.tanh(lin(concat_feat, "fc1"))


def _fused_entry(*refs):
    n_mat = len(_MAT_W)
    n_row = len(_ROW_W)
    qlast_ref, his_ref, rcnn_ref, qt_ref, qet_ref = refs[:5]
    rest = refs[5:]
    w_refs = rest[:n_mat]
    b_refs = rest[n_mat:2 * n_mat]
    rw_refs = rest[2 * n_mat:2 * n_mat + n_row]
    rb_refs = rest[2 * n_mat + n_row:2 * n_mat + 2 * n_row]
    out_ref = rest[-1]
    _fused((qlast_ref, his_ref, rcnn_ref, qt_ref, qet_ref,
            w_refs, b_refs, rw_refs, rb_refs, out_ref))


@functools.partial(jax.jit, static_argnames=("interpret",))
def _run(qlast, his, rcnn, qt, qet, mats, mbias, rows, rbias, interpret=False):
    B = qlast.shape[0]
    bB = 32
    grid = (B // bB,)

    def bspec(block, imap):
        return pl.BlockSpec(block, imap)

    const = lambda *z: (lambda i: tuple(0 for _ in z))
    in_specs = [
        bspec((bB, _NHID), lambda i: (i, 0)),
        bspec((bB, _RND, _NHID), lambda i: (i, 0, 0)),
        bspec((bB, _K, _NHID), lambda i: (i, 0, 0)),
        bspec((bB, _L, _NHID), lambda i: (i, 0, 0)),
        bspec((bB, _L, _NHID), lambda i: (i, 0, 0)),
    ]
    for a in list(mats) + list(mbias) + list(rows) + list(rbias):
        in_specs.append(bspec(a.shape, const(*a.shape)))
    out = pl.pallas_call(
        _fused_entry,
        grid=grid,
        in_specs=in_specs,
        out_specs=pl.BlockSpec((bB, _NHID), lambda i: (i, 0)),
        out_shape=jax.ShapeDtypeStruct((B, _NHID), jnp.float32),
        compiler_params=pltpu.CompilerParams(
            dimension_semantics=("arbitrary",)),
        interpret=interpret,
    )(qlast, his, rcnn, qt, qet, *mats, *mbias, *rows, *rbias)
    return out


def kernel(ques_feat, his_feat, rcnn_feat, ques_emb, params, rnd,
           interpret=False):
    p = params
    qlast = ques_feat[-1]
    qt = jnp.transpose(ques_feat, (1, 0, 2))
    qet = jnp.transpose(ques_emb, (1, 0, 2))
    mats = tuple(p[n + "_w"].T for n in _MAT_W)
    mbias = tuple(p[n + "_b"].reshape(1, -1) for n in _MAT_W)
    rows = tuple(p[n + "_w"].reshape(1, -1) for n in _ROW_W)
    rbias = tuple(p[n + "_b"].reshape(1, 1) for n in _ROW_W)
    return _run(qlast, his_feat, rcnn_feat, qt, qet,
                mats, mbias, rows, rbias, interpret=interpret)


# speedup vs baseline: 15.1081x; 1.0055x over previous
import functools

import jax
import jax.numpy as jnp
from jax.experimental import pallas as pl
from jax.experimental.pallas import tpu as pltpu

_NHID = 512
_L = 20
_RND = 10
_K = 36
_NB = 8

_MAT_W = ["Wq_1", "Wh_1", "ref_att", "ref_att2", "ref_att3",
          "W3", "W4", "W5", "W6", "W7", "W8", "W9", "W10", "fc1"]
_ROW_W = ["Wa_1", "Wqt", "Wqt2", "Wqt3", "W11"]
_BF16_W = {"W3", "W5", "W6", "W8", "W9"}


def _bmm(a, b):
    return jax.lax.dot_general(
        a, b, (((2,), (1,)), ((0,), (0,))), preferred_element_type=jnp.float32)


def _top8_weights(belief):
    neg = jnp.float32(-1e30)
    iota = jax.lax.broadcasted_iota(jnp.int32, belief.shape, 2)
    vals = belief
    mask = jnp.zeros(belief.shape, jnp.bool_)
    for _ in range(_NB):
        m = jnp.max(vals, axis=-1, keepdims=True)
        cand = vals >= m
        idx = jnp.min(jnp.where(cand, iota, _K), axis=-1, keepdims=True)
        one = iota == idx
        mask = jnp.logical_or(mask, one)
        vals = jnp.where(one, neg, vals)
    mx = jnp.max(belief, axis=-1, keepdims=True)
    e = jnp.where(mask, jnp.exp(belief - mx), 0.0)
    return e / jnp.sum(e, axis=-1, keepdims=True)


def _fused(refs):
    (qlast_ref, his_ref, rcnn_ref, qt_ref, qet_ref, w_refs, b_refs,
     rw_refs, rb_refs, out_ref) = refs
    f32 = jnp.float32
    qlast = qlast_ref[...]
    his = his_ref[...]
    rcnn = rcnn_ref[...]
    qt = qt_ref[...]
    qet = qet_ref[...]
    bB = qlast.shape[0]

    W = {n: w_refs[i][...] for i, n in enumerate(_MAT_W)}
    Bv = {n: b_refs[i][...] for i, n in enumerate(_MAT_W)}
    RW = {n: rw_refs[i][...] for i, n in enumerate(_ROW_W)}
    RB = {n: rb_refs[i][...] for i, n in enumerate(_ROW_W)}

    def lin(x, n):
        w = W[n]
        if n in _BF16_W:
            x = x.astype(jnp.bfloat16)
        return jnp.dot(x, w, preferred_element_type=f32) + Bv[n]

    def rowlin3(x, n):
        return jnp.sum(x * RW[n][None, :, :], axis=-1) + RB[n][0, 0]

    def softmax(x):
        m = jnp.max(x, axis=-1, keepdims=True)
        e = jnp.exp(x - m)
        return e / jnp.sum(e, axis=-1, keepdims=True)

    q1 = lin(qlast, "Wq_1")
    he = lin(his.reshape(bB * _RND, _NHID), "Wh_1").reshape(bB, _RND, _NHID)
    att1 = jnp.tanh(he + q1[:, None, :])
    haw = softmax(rowlin3(att1, "Wa_1"))
    h_emb = jnp.sum(haw[:, :, None] * his, axis=1)
    h_exp = jnp.broadcast_to(h_emb[:, None, :], (bB, _K, _NHID))

    def q_ctx(ref_name, wqt_name):
        qn = jax.nn.sigmoid(
            lin(qt.reshape(bB * _L, _NHID), ref_name)).reshape(bB, _L, _NHID)
        nrm = jnp.sqrt(jnp.sum(qn * qn, axis=-1, keepdims=True))
        qn = qn / jnp.maximum(nrm, 1e-12)
        at = softmax(rowlin3(qn, wqt_name))
        return jnp.sum(at[:, :, None] * qet, axis=1)

    def round_fn(g, qc):
        gf = g.reshape(bB * _K, 2 * _NHID)
        mes_b = (lin(gf, "W3").reshape(bB, _K, _NHID)
                 * lin(qc, "W4")[:, None, :])
        g5 = lin(gf, "W5").reshape(bB, _K, _NHID)
        belief = _bmm(g5, jnp.transpose(mes_b, (0, 2, 1)))
        mes = (lin(gf, "W6").reshape(bB, _K, _NHID)
               * lin(qc, "W7")[:, None, :])
        w8 = _top8_weights(belief)
        return _bmm(w8, mes)

    qc1 = q_ctx("ref_att", "Wqt")
    sm1 = round_fn(jnp.concatenate((rcnn, h_exp), axis=2), qc1)
    ctx1 = lin(jnp.concatenate((h_exp, sm1), axis=2).reshape(bB * _K, 2 * _NHID),
               "W8").reshape(bB, _K, _NHID)

    qc2 = q_ctx("ref_att2", "Wqt2")
    sm2 = round_fn(jnp.concatenate((rcnn, ctx1), axis=2), qc2)
    ctx2 = lin(jnp.concatenate((ctx1, sm2), axis=2).reshape(bB * _K, 2 * _NHID),
               "W8").reshape(bB, _K, _NHID)

    qc3 = q_ctx("ref_att3", "Wqt3")
    sm3 = round_fn(jnp.concatenate((rcnn, ctx2), axis=2), qc3)
    ctx3 = lin(jnp.concatenate((ctx2, sm3), axis=2).reshape(bB * _K, 2 * _NHID),
               "W8").reshape(bB, _K, _NHID)

    graph4 = jnp.concatenate((rcnn, ctx3), axis=2)
    g2 = lin(graph4.reshape(bB * _K, 2 * _NHID), "W9").reshape(bB, _K, _NHID)
    qe2 = lin(qlast, "W10")
    attg = jnp.tanh(g2 + qe2[:, None, :])
    gatt = softmax(rowlin3(attg, "W11"))
    graph_emb = jnp.sum(gatt[:, :, None] * graph4, axis=1)
    concat_feat = jnp.concatenate((graph_emb, qlast, h_emb), axis=1)
    out_ref[...] = jnp.tanh(lin(concat_feat, "fc1"))


def _fused_entry(*refs):
    n_mat = len(_MAT_W)
    n_row = len(_ROW_W)
    qlast_ref, his_ref, rcnn_ref, qt_ref, qet_ref = refs[:5]
    rest = refs[5:]
    w_refs = rest[:n_mat]
    b_refs = rest[n_mat:2 * n_mat]
    rw_refs = rest[2 * n_mat:2 * n_mat + n_row]
    rb_refs = rest[2 * n_mat + n_row:2 * n_mat + 2 * n_row]
    out_ref = rest[-1]
    _fused((qlast_ref, his_ref, rcnn_ref, qt_ref, qet_ref,
            w_refs, b_refs, rw_refs, rb_refs, out_ref))


@functools.partial(jax.jit, static_argnames=("interpret",))
def _run(qlast, his, rcnn, qt, qet, mats, mbias, rows, rbias, interpret=False):
    B = qlast.shape[0]
    bB = 32
    grid = (B // bB,)

    def bspec(block, imap):
        return pl.BlockSpec(block, imap)

    const = lambda *z: (lambda i: tuple(0 for _ in z))
    in_specs = [
        bspec((bB, _NHID), lambda i: (i, 0)),
        bspec((bB, _RND, _NHID), lambda i: (i, 0, 0)),
        bspec((bB, _K, _NHID), lambda i: (i, 0, 0)),
        bspec((bB, _L, _NHID), lambda i: (i, 0, 0)),
        bspec((bB, _L, _NHID), lambda i: (i, 0, 0)),
    ]
    for a in list(mats) + list(mbias) + list(rows) + list(rbias):
        in_specs.append(bspec(a.shape, const(*a.shape)))
    out = pl.pallas_call(
        _fused_entry,
        grid=grid,
        in_specs=in_specs,
        out_specs=pl.BlockSpec((bB, _NHID), lambda i: (i, 0)),
        out_shape=jax.ShapeDtypeStruct((B, _NHID), jnp.float32),
        compiler_params=pltpu.CompilerParams(
            dimension_semantics=("arbitrary",)),
        interpret=interpret,
    )(qlast, his, rcnn, qt, qet, *mats, *mbias, *rows, *rbias)
    return out


def kernel(ques_feat, his_feat, rcnn_feat, ques_emb, params, rnd,
           interpret=False):
    p = params
    qlast = ques_feat[-1]
    qt = jnp.transpose(ques_feat, (1, 0, 2))
    qet = jnp.transpose(ques_emb, (1, 0, 2))
    mats = tuple(p[n + "_w"].T.astype(jnp.bfloat16) if n in _BF16_W
                 else p[n + "_w"].T for n in _MAT_W)
    mbias = tuple(p[n + "_b"].reshape(1, -1) for n in _MAT_W)
    rows = tuple(p[n + "_w"].reshape(1, -1) for n in _ROW_W)
    rbias = tuple(p[n + "_b"].reshape(1, 1) for n in _ROW_W)
    return _run(qlast, his_feat, rcnn_feat, qt, qet,
                mats, mbias, rows, rbias, interpret=interpret)


# speedup vs baseline: 15.1183x; 1.0007x over previous
import functools

import jax
import jax.numpy as jnp
from jax.experimental import pallas as pl
from jax.experimental.pallas import tpu as pltpu

_NHID = 512
_L = 20
_RND = 10
_K = 36
_NB = 8

_MAT_W = ["Wq_1", "Wh_1", "ref_att", "ref_att2", "ref_att3",
          "W3", "W4", "W5", "W6", "W7", "W8", "W9", "W10", "fc1"]
_ROW_W = ["Wa_1", "Wqt", "Wqt2", "Wqt3", "W11"]
_BF16_W = {"W3", "W5", "W6", "W8", "W9"}


def _bmm(a, b):
    return jax.lax.dot_general(
        a, b, (((2,), (1,)), ((0,), (0,))), preferred_element_type=jnp.float32)


def _bmm_t(a, b):
    return jax.lax.dot_general(
        a, b, (((2,), (2,)), ((0,), (0,))), preferred_element_type=jnp.float32)


def _top8_weights(belief):
    neg = jnp.float32(-1e30)
    iota = jax.lax.broadcasted_iota(jnp.int32, belief.shape, 2)
    vals = belief
    mask = jnp.zeros(belief.shape, jnp.bool_)
    for _ in range(_NB):
        m = jnp.max(vals, axis=-1, keepdims=True)
        cand = vals >= m
        idx = jnp.min(jnp.where(cand, iota, _K), axis=-1, keepdims=True)
        one = iota == idx
        mask = jnp.logical_or(mask, one)
        vals = jnp.where(one, neg, vals)
    mx = jnp.max(belief, axis=-1, keepdims=True)
    e = jnp.where(mask, jnp.exp(belief - mx), 0.0)
    return e / jnp.sum(e, axis=-1, keepdims=True)


def _fused(refs):
    (qlast_ref, his_ref, rcnn_ref, qt_ref, qet_ref, w_refs, b_refs,
     rw_refs, rb_refs, out_ref) = refs
    f32 = jnp.float32
    qlast = qlast_ref[...]
    his = his_ref[...]
    rcnn = rcnn_ref[...]
    qt = qt_ref[...]
    qet = qet_ref[...]
    bB = qlast.shape[0]

    W = {n: w_refs[i][...] for i, n in enumerate(_MAT_W)}
    Bv = {n: b_refs[i][...] for i, n in enumerate(_MAT_W)}
    RW = {n: rw_refs[i][...] for i, n in enumerate(_ROW_W)}
    RB = {n: rb_refs[i][...] for i, n in enumerate(_ROW_W)}

    def lin(x, n):
        w = W[n]
        if n in _BF16_W:
            x = x.astype(jnp.bfloat16)
        return jnp.dot(x, w, preferred_element_type=f32) + Bv[n]

    def rowlin3(x, n):
        return jnp.sum(x * RW[n][None, :, :], axis=-1) + RB[n][0, 0]

    def softmax(x):
        m = jnp.max(x, axis=-1, keepdims=True)
        e = jnp.exp(x - m)
        return e / jnp.sum(e, axis=-1, keepdims=True)

    q1 = lin(qlast, "Wq_1")
    he = lin(his.reshape(bB * _RND, _NHID), "Wh_1").reshape(bB, _RND, _NHID)
    att1 = jnp.tanh(he + q1[:, None, :])
    haw = softmax(rowlin3(att1, "Wa_1"))
    h_emb = jnp.sum(haw[:, :, None] * his, axis=1)
    h_exp = jnp.broadcast_to(h_emb[:, None, :], (bB, _K, _NHID))

    def q_ctx(ref_name, wqt_name):
        qn = jax.nn.sigmoid(
            lin(qt.reshape(bB * _L, _NHID), ref_name)).reshape(bB, _L, _NHID)
        nrm = jnp.sqrt(jnp.sum(qn * qn, axis=-1, keepdims=True))
        qn = qn / jnp.maximum(nrm, 1e-12)
        at = softmax(rowlin3(qn, wqt_name))
        return jnp.sum(at[:, :, None] * qet, axis=1)

    def round_fn(g, qc):
        gf = g.reshape(bB * _K, 2 * _NHID)
        mes_b = (lin(gf, "W3").reshape(bB, _K, _NHID)
                 * lin(qc, "W4")[:, None, :])
        g5 = lin(gf, "W5").reshape(bB, _K, _NHID)
        belief = _bmm_t(g5.astype(jnp.bfloat16),
                        mes_b.astype(jnp.bfloat16))
        mes = (lin(gf, "W6").reshape(bB, _K, _NHID)
               * lin(qc, "W7")[:, None, :])
        w8 = _top8_weights(belief)
        return _bmm(w8, mes)

    qc1 = q_ctx("ref_att", "Wqt")
    sm1 = round_fn(jnp.concatenate((rcnn, h_exp), axis=2), qc1)
    ctx1 = lin(jnp.concatenate((h_exp, sm1), axis=2).reshape(bB * _K, 2 * _NHID),
               "W8").reshape(bB, _K, _NHID)

    qc2 = q_ctx("ref_att2", "Wqt2")
    sm2 = round_fn(jnp.concatenate((rcnn, ctx1), axis=2), qc2)
    ctx2 = lin(jnp.concatenate((ctx1, sm2), axis=2).reshape(bB * _K, 2 * _NHID),
               "W8").reshape(bB, _K, _NHID)

    qc3 = q_ctx("ref_att3", "Wqt3")
    sm3 = round_fn(jnp.concatenate((rcnn, ctx2), axis=2), qc3)
    ctx3 = lin(jnp.concatenate((ctx2, sm3), axis=2).reshape(bB * _K, 2 * _NHID),
               "W8").reshape(bB, _K, _NHID)

    graph4 = jnp.concatenate((rcnn, ctx3), axis=2)
    g2 = lin(graph4.reshape(bB * _K, 2 * _NHID), "W9").reshape(bB, _K, _NHID)
    qe2 = lin(qlast, "W10")
    attg = jnp.tanh(g2 + qe2[:, None, :])
    gatt = softmax(rowlin3(attg, "W11"))
    graph_emb = jnp.sum(gatt[:, :, None] * graph4, axis=1)
    concat_feat = jnp.concatenate((graph_emb, qlast, h_emb), axis=1)
    out_ref[...] = jnp.tanh(lin(concat_feat, "fc1"))


def _fused_entry(*refs):
    n_mat = len(_MAT_W)
    n_row = len(_ROW_W)
    qlast_ref, his_ref, rcnn_ref, qt_ref, qet_ref = refs[:5]
    rest = refs[5:]
    w_refs = rest[:n_mat]
    b_refs = rest[n_mat:2 * n_mat]
    rw_refs = rest[2 * n_mat:2 * n_mat + n_row]
    rb_refs = rest[2 * n_mat + n_row:2 * n_mat + 2 * n_row]
    out_ref = rest[-1]
    _fused((qlast_ref, his_ref, rcnn_ref, qt_ref, qet_ref,
            w_refs, b_refs, rw_refs, rb_refs, out_ref))


@functools.partial(jax.jit, static_argnames=("interpret",))
def _run(qlast, his, rcnn, qt, qet, mats, mbias, rows, rbias, interpret=False):
    B = qlast.shape[0]
    bB = 32
    grid = (B // bB,)

    def bspec(block, imap):
        return pl.BlockSpec(block, imap)

    const = lambda *z: (lambda i: tuple(0 for _ in z))
    in_specs = [
        bspec((bB, _NHID), lambda i: (i, 0)),
        bspec((bB, _RND, _NHID), lambda i: (i, 0, 0)),
        bspec((bB, _K, _NHID), lambda i: (i, 0, 0)),
        bspec((bB, _L, _NHID), lambda i: (i, 0, 0)),
        bspec((bB, _L, _NHID), lambda i: (i, 0, 0)),
    ]
    for a in list(mats) + list(mbias) + list(rows) + list(rbias):
        in_specs.append(bspec(a.shape, const(*a.shape)))
    out = pl.pallas_call(
        _fused_entry,
        grid=grid,
        in_specs=in_specs,
        out_specs=pl.BlockSpec((bB, _NHID), lambda i: (i, 0)),
        out_shape=jax.ShapeDtypeStruct((B, _NHID), jnp.float32),
        compiler_params=pltpu.CompilerParams(
            dimension_semantics=("arbitrary",)),
        interpret=interpret,
    )(qlast, his, rcnn, qt, qet, *mats, *mbias, *rows, *rbias)
    return out


def kernel(ques_feat, his_feat, rcnn_feat, ques_emb, params, rnd,
           interpret=False):
    p = params
    qlast = ques_feat[-1]
    qt = jnp.transpose(ques_feat, (1, 0, 2))
    qet = jnp.transpose(ques_emb, (1, 0, 2))
    mats = tuple(p[n + "_w"].T.astype(jnp.bfloat16) if n in _BF16_W
                 else p[n + "_w"].T for n in _MAT_W)
    mbias = tuple(p[n + "_b"].reshape(1, -1) for n in _MAT_W)
    rows = tuple(p[n + "_w"].reshape(1, -1) for n in _ROW_W)
    rbias = tuple(p[n + "_b"].reshape(1, 1) for n in _ROW_W)
    return _run(qlast, his_feat, rcnn_feat, qt, qet,
                mats, mbias, rows, rbias, interpret=interpret)


# speedup vs baseline: 18.9559x; 1.2538x over previous
import functools

import jax
import jax.numpy as jnp
from jax.experimental import pallas as pl
from jax.experimental.pallas import tpu as pltpu

_NHID = 512
_L = 20
_RND = 10
_K = 36
_NB = 8

_MAT_W = ["Wq_1", "Wh_1", "ref_att", "ref_att2", "ref_att3",
          "W3", "W4", "W5", "W6", "W7", "W8", "W9", "W10", "fc1"]
_ROW_W = ["Wa_1", "Wqt", "Wqt2", "Wqt3", "W11"]
_BF16_W = {"W3", "W5", "W6", "W8", "W9"}


def _bmm(a, b):
    return jax.lax.dot_general(
        a, b, (((2,), (1,)), ((0,), (0,))), preferred_element_type=jnp.float32)


def _bmm_t(a, b):
    return jax.lax.dot_general(
        a, b, (((2,), (2,)), ((0,), (0,))), preferred_element_type=jnp.float32)


def _top8_weights(belief):
    neg = jnp.float32(-1e30)
    vals = belief
    mx = None
    for t in range(_NB):
        m = jnp.max(vals, axis=-1, keepdims=True)
        if t == 0:
            mx = m
        vals = jnp.where(vals >= m, neg, vals)
    e = jnp.where(vals == belief, 0.0, jnp.exp(belief - mx))
    return e / jnp.sum(e, axis=-1, keepdims=True)


def _fused(refs):
    (qlast_ref, his_ref, rcnn_ref, qt_ref, qet_ref, w_refs, b_refs,
     rw_refs, rb_refs, out_ref) = refs
    f32 = jnp.float32
    qlast = qlast_ref[...]
    his = his_ref[...]
    rcnn = rcnn_ref[...]
    qt = qt_ref[...]
    qet = qet_ref[...]
    bB = qlast.shape[0]

    W = {n: w_refs[i][...] for i, n in enumerate(_MAT_W)}
    Bv = {n: b_refs[i][...] for i, n in enumerate(_MAT_W)}
    RW = {n: rw_refs[i][...] for i, n in enumerate(_ROW_W)}
    RB = {n: rb_refs[i][...] for i, n in enumerate(_ROW_W)}

    def lin(x, n):
        w = W[n]
        if n in _BF16_W:
            x = x.astype(jnp.bfloat16)
        return jnp.dot(x, w, preferred_element_type=f32) + Bv[n]

    def rowlin3(x, n):
        return jnp.sum(x * RW[n][None, :, :], axis=-1) + RB[n][0, 0]

    def softmax(x):
        m = jnp.max(x, axis=-1, keepdims=True)
        e = jnp.exp(x - m)
        return e / jnp.sum(e, axis=-1, keepdims=True)

    q1 = lin(qlast, "Wq_1")
    he = lin(his.reshape(bB * _RND, _NHID), "Wh_1").reshape(bB, _RND, _NHID)
    att1 = jnp.tanh(he + q1[:, None, :])
    haw = softmax(rowlin3(att1, "Wa_1"))
    h_emb = jnp.sum(haw[:, :, None] * his, axis=1)
    h_exp = jnp.broadcast_to(h_emb[:, None, :], (bB, _K, _NHID))

    def q_ctx(ref_name, wqt_name):
        qn = jax.nn.sigmoid(
            lin(qt.reshape(bB * _L, _NHID), ref_name)).reshape(bB, _L, _NHID)
        nrm = jnp.sqrt(jnp.sum(qn * qn, axis=-1, keepdims=True))
        qn = qn / jnp.maximum(nrm, 1e-12)
        at = softmax(rowlin3(qn, wqt_name))
        return jnp.sum(at[:, :, None] * qet, axis=1)

    def round_fn(g, qc):
        gf = g.reshape(bB * _K, 2 * _NHID)
        mes_b = (lin(gf, "W3").reshape(bB, _K, _NHID)
                 * lin(qc, "W4")[:, None, :])
        g5 = lin(gf, "W5").reshape(bB, _K, _NHID)
        belief = _bmm_t(g5.astype(jnp.bfloat16),
                        mes_b.astype(jnp.bfloat16))
        mes = (lin(gf, "W6").reshape(bB, _K, _NHID)
               * lin(qc, "W7")[:, None, :])
        w8 = _top8_weights(belief)
        return _bmm(w8, mes)

    qc1 = q_ctx("ref_att", "Wqt")
    sm1 = round_fn(jnp.concatenate((rcnn, h_exp), axis=2), qc1)
    ctx1 = lin(jnp.concatenate((h_exp, sm1), axis=2).reshape(bB * _K, 2 * _NHID),
               "W8").reshape(bB, _K, _NHID)

    qc2 = q_ctx("ref_att2", "Wqt2")
    sm2 = round_fn(jnp.concatenate((rcnn, ctx1), axis=2), qc2)
    ctx2 = lin(jnp.concatenate((ctx1, sm2), axis=2).reshape(bB * _K, 2 * _NHID),
               "W8").reshape(bB, _K, _NHID)

    qc3 = q_ctx("ref_att3", "Wqt3")
    sm3 = round_fn(jnp.concatenate((rcnn, ctx2), axis=2), qc3)
    ctx3 = lin(jnp.concatenate((ctx2, sm3), axis=2).reshape(bB * _K, 2 * _NHID),
               "W8").reshape(bB, _K, _NHID)

    graph4 = jnp.concatenate((rcnn, ctx3), axis=2)
    g2 = lin(graph4.reshape(bB * _K, 2 * _NHID), "W9").reshape(bB, _K, _NHID)
    qe2 = lin(qlast, "W10")
    attg = jnp.tanh(g2 + qe2[:, None, :])
    gatt = softmax(rowlin3(attg, "W11"))
    graph_emb = jnp.sum(gatt[:, :, None] * graph4, axis=1)
    concat_feat = jnp.concatenate((graph_emb, qlast, h_emb), axis=1)
    out_ref[...] = jnp.tanh(lin(concat_feat, "fc1"))


def _fused_entry(*refs):
    n_mat = len(_MAT_W)
    n_row = len(_ROW_W)
    qlast_ref, his_ref, rcnn_ref, qt_ref, qet_ref = refs[:5]
    rest = refs[5:]
    w_refs = rest[:n_mat]
    b_refs = rest[n_mat:2 * n_mat]
    rw_refs = rest[2 * n_mat:2 * n_mat + n_row]
    rb_refs = rest[2 * n_mat + n_row:2 * n_mat + 2 * n_row]
    out_ref = rest[-1]
    _fused((qlast_ref, his_ref, rcnn_ref, qt_ref, qet_ref,
            w_refs, b_refs, rw_refs, rb_refs, out_ref))


@functools.partial(jax.jit, static_argnames=("interpret",))
def _run(qlast, his, rcnn, qt, qet, mats, mbias, rows, rbias, interpret=False):
    B = qlast.shape[0]
    bB = 32
    grid = (B // bB,)

    def bspec(block, imap):
        return pl.BlockSpec(block, imap)

    const = lambda *z: (lambda i: tuple(0 for _ in z))
    in_specs = [
        bspec((bB, _NHID), lambda i: (i, 0)),
        bspec((bB, _RND, _NHID), lambda i: (i, 0, 0)),
        bspec((bB, _K, _NHID), lambda i: (i, 0, 0)),
        bspec((bB, _L, _NHID), lambda i: (i, 0, 0)),
        bspec((bB, _L, _NHID), lambda i: (i, 0, 0)),
    ]
    for a in list(mats) + list(mbias) + list(rows) + list(rbias):
        in_specs.append(bspec(a.shape, const(*a.shape)))
    out = pl.pallas_call(
        _fused_entry,
        grid=grid,
        in_specs=in_specs,
        out_specs=pl.BlockSpec((bB, _NHID), lambda i: (i, 0)),
        out_shape=jax.ShapeDtypeStruct((B, _NHID), jnp.float32),
        compiler_params=pltpu.CompilerParams(
            dimension_semantics=("arbitrary",)),
        interpret=interpret,
    )(qlast, his, rcnn, qt, qet, *mats, *mbias, *rows, *rbias)
    return out


def kernel(ques_feat, his_feat, rcnn_feat, ques_emb, params, rnd,
           interpret=False):
    p = params
    qlast = ques_feat[-1]
    qt = jnp.transpose(ques_feat, (1, 0, 2))
    qet = jnp.transpose(ques_emb, (1, 0, 2))
    mats = tuple(p[n + "_w"].T.astype(jnp.bfloat16) if n in _BF16_W
                 else p[n + "_w"].T for n in _MAT_W)
    mbias = tuple(p[n + "_b"].reshape(1, -1) for n in _MAT_W)
    rows = tuple(p[n + "_w"].reshape(1, -1) for n in _ROW_W)
    rbias = tuple(p[n + "_b"].reshape(1, 1) for n in _ROW_W)
    return _run(qlast, his_feat, rcnn_feat, qt, qet,
                mats, mbias, rows, rbias, interpret=interpret)


# speedup vs baseline: 19.0960x; 1.0074x over previous
import functools

import jax
import jax.numpy as jnp
from jax.experimental import pallas as pl
from jax.experimental.pallas import tpu as pltpu

_NHID = 512
_L = 20
_RND = 10
_K = 36
_NB = 8

_PIECES = ["Wq_1t", "Wh_1t", "ref_att_t", "ref_att2_t", "ref_att3_t",
           "W3r", "W3c", "W4t", "W5r", "W5c", "W6r", "W6c", "W7t",
           "W8a", "W8b", "W9r", "W9c", "W10t", "F0", "F1", "F2", "F3"]
_BF16 = {"W3r", "W3c", "W5r", "W5c", "W6r", "W6c", "W8a", "W8b",
         "W9r", "W9c"}
_BIASES = ["Wq_1", "Wh_1", "ref_att", "ref_att2", "ref_att3",
           "W3", "W4", "W5", "W6", "W7", "W8", "W9", "W10", "fc1"]
_ROW_W = ["Wa_1", "Wqt", "Wqt2", "Wqt3", "W11"]


def _bmm(a, b):
    return jax.lax.dot_general(
        a, b, (((2,), (1,)), ((0,), (0,))), preferred_element_type=jnp.float32)


def _bmm_t(a, b):
    return jax.lax.dot_general(
        a, b, (((2,), (2,)), ((0,), (0,))), preferred_element_type=jnp.float32)


def _top8_weights(belief):
    neg = jnp.float32(-1e30)
    vals = belief
    mx = None
    for t in range(_NB):
        m = jnp.max(vals, axis=-1, keepdims=True)
        if t == 0:
            mx = m
        vals = jnp.where(vals >= m, neg, vals)
    e = jnp.where(vals == belief, 0.0, jnp.exp(belief - mx))
    return e / jnp.sum(e, axis=-1, keepdims=True)


def _fused(refs):
    (qlast_ref, his_ref, rcnn_ref, qt_ref, qet_ref, w_refs, b_refs,
     rw_refs, rb_refs, out_ref) = refs
    f32 = jnp.float32
    qlast = qlast_ref[...]
    his = his_ref[...]
    rcnn = rcnn_ref[...]
    qt = qt_ref[...]
    qet = qet_ref[...]
    bB = qlast.shape[0]

    W = {n: w_refs[i][...] for i, n in enumerate(_PIECES)}
    Bv = {n: b_refs[i][...] for i, n in enumerate(_BIASES)}
    RW = {n: rw_refs[i][...] for i, n in enumerate(_ROW_W)}
    RB = {n: rb_refs[i][...] for i, n in enumerate(_ROW_W)}

    def mm(x, n):
        w = W[n]
        if n in _BF16:
            x = x.astype(jnp.bfloat16)
        return jnp.dot(x, w, preferred_element_type=f32)

    def rowlin3(x, n):
        return jnp.sum(x * RW[n][None, :, :], axis=-1) + RB[n][0, 0]

    def softmax(x):
        m = jnp.max(x, axis=-1, keepdims=True)
        e = jnp.exp(x - m)
        return e / jnp.sum(e, axis=-1, keepdims=True)

    q1 = mm(qlast, "Wq_1t") + Bv["Wq_1"]
    he = (mm(his.reshape(bB * _RND, _NHID), "Wh_1t")
          + Bv["Wh_1"]).reshape(bB, _RND, _NHID)
    att1 = jnp.tanh(he + q1[:, None, :])
    haw = softmax(rowlin3(att1, "Wa_1"))
    h_emb = jnp.sum(haw[:, :, None] * his, axis=1)

    def q_ctx(ref_name, wqt_name, bias_name):
        qn = jax.nn.sigmoid(
            mm(qt.reshape(bB * _L, _NHID), ref_name)
            + Bv[bias_name]).reshape(bB, _L, _NHID)
        nrm = jnp.sqrt(jnp.sum(qn * qn, axis=-1, keepdims=True))
        qn = qn / jnp.maximum(nrm, 1e-12)
        at = softmax(rowlin3(qn, wqt_name))
        return jnp.sum(at[:, :, None] * qet, axis=1)

    rcnnf = rcnn.reshape(bB * _K, _NHID)
    r3 = mm(rcnnf, "W3r").reshape(bB, _K, _NHID)
    r5 = mm(rcnnf, "W5r").reshape(bB, _K, _NHID)
    r6 = mm(rcnnf, "W6r").reshape(bB, _K, _NHID)

    def round_fn(c3, c5, c6, qc):
        mes_b = ((r3 + c3 + Bv["W3"][None, :, :])
                 * (mm(qc, "W4t") + Bv["W4"])[:, None, :])
        g5 = r5 + c5 + Bv["W5"][None, :, :]
        belief = _bmm_t(g5.astype(jnp.bfloat16),
                        mes_b.astype(jnp.bfloat16))
        mes = ((r6 + c6 + Bv["W6"][None, :, :])
               * (mm(qc, "W7t") + Bv["W7"])[:, None, :])
        w8 = _top8_weights(belief)
        return _bmm(w8, mes)

    qc1 = q_ctx("ref_att_t", "Wqt", "ref_att")
    sm1 = round_fn(mm(h_emb, "W3c")[:, None, :],
                   mm(h_emb, "W5c")[:, None, :],
                   mm(h_emb, "W6c")[:, None, :], qc1)
    ctx1 = ((mm(h_emb, "W8a") + Bv["W8"])[:, None, :]
            + mm(sm1.reshape(bB * _K, _NHID), "W8b").reshape(bB, _K, _NHID))

    qc2 = q_ctx("ref_att2_t", "Wqt2", "ref_att2")
    ctx1f = ctx1.reshape(bB * _K, _NHID)
    sm2 = round_fn(mm(ctx1f, "W3c").reshape(bB, _K, _NHID),
                   mm(ctx1f, "W5c").reshape(bB, _K, _NHID),
                   mm(ctx1f, "W6c").reshape(bB, _K, _NHID), qc2)
    ctx2 = (mm(ctx1f, "W8a").reshape(bB, _K, _NHID)
            + mm(sm2.reshape(bB * _K, _NHID), "W8b").reshape(bB, _K, _NHID)
            + Bv["W8"][None, :, :])

    qc3 = q_ctx("ref_att3_t", "Wqt3", "ref_att3")
    ctx2f = ctx2.reshape(bB * _K, _NHID)
    sm3 = round_fn(mm(ctx2f, "W3c").reshape(bB, _K, _NHID),
                   mm(ctx2f, "W5c").reshape(bB, _K, _NHID),
                   mm(ctx2f, "W6c").reshape(bB, _K, _NHID), qc3)
    ctx3 = (mm(ctx2f, "W8a").reshape(bB, _K, _NHID)
            + mm(sm3.reshape(bB * _K, _NHID), "W8b").reshape(bB, _K, _NHID)
            + Bv["W8"][None, :, :])

    ctx3f = ctx3.reshape(bB * _K, _NHID)
    g2 = (mm(rcnnf, "W9r") + mm(ctx3f, "W9c")
          + Bv["W9"]).reshape(bB, _K, _NHID)
    qe2 = mm(qlast, "W10t") + Bv["W10"]
    attg = jnp.tanh(g2 + qe2[:, None, :])
    gatt = softmax(rowlin3(attg, "W11"))
    ge_r = jnp.sum(gatt[:, :, None] * rcnn, axis=1)
    ge_c = jnp.sum(gatt[:, :, None] * ctx3, axis=1)
    out = (mm(ge_r, "F0") + mm(ge_c, "F1") + mm(qlast, "F2")
           + mm(h_emb, "F3") + Bv["fc1"])
    out_ref[...] = jnp.tanh(out)


def _fused_entry(*refs):
    np_, nb, nr = len(_PIECES), len(_BIASES), len(_ROW_W)
    qlast_ref, his_ref, rcnn_ref, qt_ref, qet_ref = refs[:5]
    rest = refs[5:]
    w_refs = rest[:np_]
    b_refs = rest[np_:np_ + nb]
    rw_refs = rest[np_ + nb:np_ + nb + nr]
    rb_refs = rest[np_ + nb + nr:np_ + nb + nr + nr]
    out_ref = rest[-1]
    _fused((qlast_ref, his_ref, rcnn_ref, qt_ref, qet_ref,
            w_refs, b_refs, rw_refs, rb_refs, out_ref))


@functools.partial(jax.jit, static_argnames=("interpret",))
def _run(qlast, his, rcnn, qt, qet, pieces, biases, rows, rbias,
         interpret=False):
    B = qlast.shape[0]
    bB = 32
    grid = (B // bB,)

    const = lambda shape: (lambda i: tuple(0 for _ in shape))
    in_specs = [
        pl.BlockSpec((bB, _NHID), lambda i: (i, 0)),
        pl.BlockSpec((bB, _RND, _NHID), lambda i: (i, 0, 0)),
        pl.BlockSpec((bB, _K, _NHID), lambda i: (i, 0, 0)),
        pl.BlockSpec((bB, _L, _NHID), lambda i: (i, 0, 0)),
        pl.BlockSpec((bB, _L, _NHID), lambda i: (i, 0, 0)),
    ]
    for a in list(pieces) + list(biases) + list(rows) + list(rbias):
        in_specs.append(pl.BlockSpec(a.shape, const(a.shape)))
    out = pl.pallas_call(
        _fused_entry,
        grid=grid,
        in_specs=in_specs,
        out_specs=pl.BlockSpec((bB, _NHID), lambda i: (i, 0)),
        out_shape=jax.ShapeDtypeStruct((B, _NHID), jnp.float32),
        compiler_params=pltpu.CompilerParams(
            dimension_semantics=("arbitrary",)),
        interpret=interpret,
    )(qlast, his, rcnn, qt, qet, *pieces, *biases, *rows, *rbias)
    return out


def kernel(ques_feat, his_feat, rcnn_feat, ques_emb, params, rnd,
           interpret=False):
    p = params
    qlast = ques_feat[-1]
    qt = jnp.transpose(ques_feat, (1, 0, 2))
    qet = jnp.transpose(ques_emb, (1, 0, 2))

    def t(n):
        return p[n + "_w"].T

    pc = {
        "Wq_1t": t("Wq_1"), "Wh_1t": t("Wh_1"),
        "ref_att_t": t("ref_att"), "ref_att2_t": t("ref_att2"),
        "ref_att3_t": t("ref_att3"),
        "W3r": t("W3")[:_NHID], "W3c": t("W3")[_NHID:],
        "W4t": t("W4"),
        "W5r": t("W5")[:_NHID], "W5c": t("W5")[_NHID:],
        "W6r": t("W6")[:_NHID], "W6c": t("W6")[_NHID:],
        "W7t": t("W7"),
        "W8a": t("W8")[:_NHID], "W8b": t("W8")[_NHID:],
        "W9r": t("W9")[:_NHID], "W9c": t("W9")[_NHID:],
        "W10t": t("W10"),
        "F0": t("fc1")[:_NHID], "F1": t("fc1")[_NHID:2 * _NHID],
        "F2": t("fc1")[2 * _NHID:3 * _NHID], "F3": t("fc1")[3 * _NHID:],
    }
    pieces = tuple(pc[n].astype(jnp.bfloat16) if n in _BF16 else pc[n]
                   for n in _PIECES)
    biases = tuple(p[n + "_b"].reshape(1, -1) for n in _BIASES)
    rows = tuple(p[n + "_w"].reshape(1, -1) for n in _ROW_W)
    rbias = tuple(p[n + "_b"].reshape(1, 1) for n in _ROW_W)
    return _run(qlast, his_feat, rcnn_feat, qt, qet,
                pieces, biases, rows, rbias, interpret=interpret)


# speedup vs baseline: 19.1987x; 1.0054x over previous
import functools

import jax
import jax.numpy as jnp
from jax.experimental import pallas as pl
from jax.experimental.pallas import tpu as pltpu

_NHID = 512
_L = 20
_RND = 10
_K = 36
_NB = 8

_PIECES = ["Wq_1t", "Wh_1t", "ref_att_t", "ref_att2_t", "ref_att3_t",
           "W3r", "W3c", "W4t", "W5r", "W5c", "W6r", "W6c", "W7t",
           "W8a", "W8b", "W9r", "W9c", "W10t", "F0", "F1", "F2", "F3"]
_BF16 = set()
_BIASES = ["Wq_1", "Wh_1", "ref_att", "ref_att2", "ref_att3",
           "W3", "W4", "W5", "W6", "W7", "W8", "W9", "W10", "fc1"]
_ROW_W = ["Wa_1", "Wqt", "Wqt2", "Wqt3", "W11"]


def _bmm(a, b):
    return jax.lax.dot_general(
        a, b, (((2,), (1,)), ((0,), (0,))), preferred_element_type=jnp.float32)


def _bmm_t(a, b):
    return jax.lax.dot_general(
        a, b, (((2,), (2,)), ((0,), (0,))), preferred_element_type=jnp.float32)


def _top8_weights(belief):
    neg = jnp.float32(-1e30)
    vals = belief
    mx = None
    for t in range(_NB):
        m = jnp.max(vals, axis=-1, keepdims=True)
        if t == 0:
            mx = m
        vals = jnp.where(vals >= m, neg, vals)
    e = jnp.where(vals == belief, 0.0, jnp.exp(belief - mx))
    return e / jnp.sum(e, axis=-1, keepdims=True)


def _fused(refs):
    (qlast_ref, his_ref, rcnn_ref, qt_ref, qet_ref, w_refs, b_refs,
     rw_refs, rb_refs, out_ref) = refs
    f32 = jnp.float32
    qlast = qlast_ref[...]
    his = his_ref[...]
    rcnn = rcnn_ref[...]
    qt = qt_ref[...]
    qet = qet_ref[...]
    bB = qlast.shape[0]

    W = {n: w_refs[i][...] for i, n in enumerate(_PIECES)}
    Bv = {n: b_refs[i][...] for i, n in enumerate(_BIASES)}
    RW = {n: rw_refs[i][...] for i, n in enumerate(_ROW_W)}
    RB = {n: rb_refs[i][...] for i, n in enumerate(_ROW_W)}

    def mm(x, n):
        w = W[n]
        if n in _BF16:
            x = x.astype(jnp.bfloat16)
        return jnp.dot(x, w, preferred_element_type=f32)

    def rowlin3(x, n):
        return jnp.sum(x * RW[n][None, :, :], axis=-1) + RB[n][0, 0]

    def softmax(x):
        m = jnp.max(x, axis=-1, keepdims=True)
        e = jnp.exp(x - m)
        return e / jnp.sum(e, axis=-1, keepdims=True)

    q1 = mm(qlast, "Wq_1t") + Bv["Wq_1"]
    he = (mm(his.reshape(bB * _RND, _NHID), "Wh_1t")
          + Bv["Wh_1"]).reshape(bB, _RND, _NHID)
    att1 = jnp.tanh(he + q1[:, None, :])
    haw = softmax(rowlin3(att1, "Wa_1"))
    h_emb = jnp.sum(haw[:, :, None] * his, axis=1)

    def q_ctx(ref_name, wqt_name, bias_name):
        qn = jax.nn.sigmoid(
            mm(qt.reshape(bB * _L, _NHID), ref_name)
            + Bv[bias_name]).reshape(bB, _L, _NHID)
        nrm = jnp.sqrt(jnp.sum(qn * qn, axis=-1, keepdims=True))
        qn = qn / jnp.maximum(nrm, 1e-12)
        at = softmax(rowlin3(qn, wqt_name))
        return jnp.sum(at[:, :, None] * qet, axis=1)

    rcnnf = rcnn.reshape(bB * _K, _NHID)
    r3 = mm(rcnnf, "W3r").reshape(bB, _K, _NHID)
    r5 = mm(rcnnf, "W5r").reshape(bB, _K, _NHID)
    r6 = mm(rcnnf, "W6r").reshape(bB, _K, _NHID)

    def round_fn(c3, c5, c6, qc):
        mes_b = ((r3 + c3 + Bv["W3"][None, :, :])
                 * (mm(qc, "W4t") + Bv["W4"])[:, None, :])
        g5 = r5 + c5 + Bv["W5"][None, :, :]
        belief = _bmm_t(g5, mes_b)
        mes = ((r6 + c6 + Bv["W6"][None, :, :])
               * (mm(qc, "W7t") + Bv["W7"])[:, None, :])
        w8 = _top8_weights(belief)
        return _bmm(w8, mes)

    qc1 = q_ctx("ref_att_t", "Wqt", "ref_att")
    sm1 = round_fn(mm(h_emb, "W3c")[:, None, :],
                   mm(h_emb, "W5c")[:, None, :],
                   mm(h_emb, "W6c")[:, None, :], qc1)
    ctx1 = ((mm(h_emb, "W8a") + Bv["W8"])[:, None, :]
            + mm(sm1.reshape(bB * _K, _NHID), "W8b").reshape(bB, _K, _NHID))

    qc2 = q_ctx("ref_att2_t", "Wqt2", "ref_att2")
    ctx1f = ctx1.reshape(bB * _K, _NHID)
    sm2 = round_fn(mm(ctx1f, "W3c").reshape(bB, _K, _NHID),
                   mm(ctx1f, "W5c").reshape(bB, _K, _NHID),
                   mm(ctx1f, "W6c").reshape(bB, _K, _NHID), qc2)
    ctx2 = (mm(ctx1f, "W8a").reshape(bB, _K, _NHID)
            + mm(sm2.reshape(bB * _K, _NHID), "W8b").reshape(bB, _K, _NHID)
            + Bv["W8"][None, :, :])

    qc3 = q_ctx("ref_att3_t", "Wqt3", "ref_att3")
    ctx2f = ctx2.reshape(bB * _K, _NHID)
    sm3 = round_fn(mm(ctx2f, "W3c").reshape(bB, _K, _NHID),
                   mm(ctx2f, "W5c").reshape(bB, _K, _NHID),
                   mm(ctx2f, "W6c").reshape(bB, _K, _NHID), qc3)
    ctx3 = (mm(ctx2f, "W8a").reshape(bB, _K, _NHID)
            + mm(sm3.reshape(bB * _K, _NHID), "W8b").reshape(bB, _K, _NHID)
            + Bv["W8"][None, :, :])

    ctx3f = ctx3.reshape(bB * _K, _NHID)
    g2 = (mm(rcnnf, "W9r") + mm(ctx3f, "W9c")
          + Bv["W9"]).reshape(bB, _K, _NHID)
    qe2 = mm(qlast, "W10t") + Bv["W10"]
    attg = jnp.tanh(g2 + qe2[:, None, :])
    gatt = softmax(rowlin3(attg, "W11"))
    ge_r = jnp.sum(gatt[:, :, None] * rcnn, axis=1)
    ge_c = jnp.sum(gatt[:, :, None] * ctx3, axis=1)
    out = (mm(ge_r, "F0") + mm(ge_c, "F1") + mm(qlast, "F2")
           + mm(h_emb, "F3") + Bv["fc1"])
    out_ref[...] = jnp.tanh(out)


def _fused_entry(*refs):
    np_, nb, nr = len(_PIECES), len(_BIASES), len(_ROW_W)
    qlast_ref, his_ref, rcnn_ref, qt_ref, qet_ref = refs[:5]
    rest = refs[5:]
    w_refs = rest[:np_]
    b_refs = rest[np_:np_ + nb]
    rw_refs = rest[np_ + nb:np_ + nb + nr]
    rb_refs = rest[np_ + nb + nr:np_ + nb + nr + nr]
    out_ref = rest[-1]
    _fused((qlast_ref, his_ref, rcnn_ref, qt_ref, qet_ref,
            w_refs, b_refs, rw_refs, rb_refs, out_ref))


@functools.partial(jax.jit, static_argnames=("interpret",))
def _run(qlast, his, rcnn, qt, qet, pieces, biases, rows, rbias,
         interpret=False):
    B = qlast.shape[0]
    bB = 32
    grid = (B // bB,)

    const = lambda shape: (lambda i: tuple(0 for _ in shape))
    in_specs = [
        pl.BlockSpec((bB, _NHID), lambda i: (i, 0)),
        pl.BlockSpec((bB, _RND, _NHID), lambda i: (i, 0, 0)),
        pl.BlockSpec((bB, _K, _NHID), lambda i: (i, 0, 0)),
        pl.BlockSpec((bB, _L, _NHID), lambda i: (i, 0, 0)),
        pl.BlockSpec((bB, _L, _NHID), lambda i: (i, 0, 0)),
    ]
    for a in list(pieces) + list(biases) + list(rows) + list(rbias):
        in_specs.append(pl.BlockSpec(a.shape, const(a.shape)))
    out = pl.pallas_call(
        _fused_entry,
        grid=grid,
        in_specs=in_specs,
        out_specs=pl.BlockSpec((bB, _NHID), lambda i: (i, 0)),
        out_shape=jax.ShapeDtypeStruct((B, _NHID), jnp.float32),
        compiler_params=pltpu.CompilerParams(
            dimension_semantics=("arbitrary",)),
        interpret=interpret,
    )(qlast, his, rcnn, qt, qet, *pieces, *biases, *rows, *rbias)
    return out


def kernel(ques_feat, his_feat, rcnn_feat, ques_emb, params, rnd,
           interpret=False):
    p = params
    qlast = ques_feat[-1]
    qt = jnp.transpose(ques_feat, (1, 0, 2))
    qet = jnp.transpose(ques_emb, (1, 0, 2))

    def t(n):
        return p[n + "_w"].T

    pc = {
        "Wq_1t": t("Wq_1"), "Wh_1t": t("Wh_1"),
        "ref_att_t": t("ref_att"), "ref_att2_t": t("ref_att2"),
        "ref_att3_t": t("ref_att3"),
        "W3r": t("W3")[:_NHID], "W3c": t("W3")[_NHID:],
        "W4t": t("W4"),
        "W5r": t("W5")[:_NHID], "W5c": t("W5")[_NHID:],
        "W6r": t("W6")[:_NHID], "W6c": t("W6")[_NHID:],
        "W7t": t("W7"),
        "W8a": t("W8")[:_NHID], "W8b": t("W8")[_NHID:],
        "W9r": t("W9")[:_NHID], "W9c": t("W9")[_NHID:],
        "W10t": t("W10"),
        "F0": t("fc1")[:_NHID], "F1": t("fc1")[_NHID:2 * _NHID],
        "F2": t("fc1")[2 * _NHID:3 * _NHID], "F3": t("fc1")[3 * _NHID:],
    }
    pieces = tuple(pc[n].astype(jnp.bfloat16) if n in _BF16 else pc[n]
                   for n in _PIECES)
    biases = tuple(p[n + "_b"].reshape(1, -1) for n in _BIASES)
    rows = tuple(p[n + "_w"].reshape(1, -1) for n in _ROW_W)
    rbias = tuple(p[n + "_b"].reshape(1, 1) for n in _ROW_W)
    return _run(qlast, his_feat, rcnn_feat, qt, qet,
                pieces, biases, rows, rbias, interpret=interpret)
